# baseline (speedup 1.0000x reference)
import functools

import jax
import jax.numpy as jnp
from jax import lax
from jax.experimental import pallas as pl
from jax.experimental.pallas import tpu as pltpu
from jax.experimental.pallas import tpu_sc as plsc

B = 4096
L = 200
D = 64
SCALE = float(D) ** 0.5
PAD = 0

_info = plsc.get_sparse_core_info()
NC = _info.num_cores
NS = _info.num_subcores
NW = NC * NS
BBLK = B // NW
NBUF = 3
LANES = 16
LT, LW = L // 8, 8
DT, DW = D // 8, 8


def _sc_embed_body(idx4_hbm, item_hbm, pos_hbm, out_hbm,
                   pos_v, idx0, idx1, idx2, g0, g1, g2, ob0, ob1, ob2,
                   isem0, isem1, isem2, gsem0, gsem1, gsem2,
                   wsem0, wsem1, wsem2):
    idxb = [idx0, idx1, idx2]
    gbuf = [g0, g1, g2]
    obuf = [ob0, ob1, ob2]
    isem = [isem0, isem1, isem2]
    gsem = [gsem0, gsem1, gsem2]
    wsem = [wsem0, wsem1, wsem2]

    w = lax.axis_index("s") * NC + lax.axis_index("c")

    pltpu.sync_copy(pos_hbm, pos_v)

    def start_idx(l, b):
        pltpu.async_copy(idx4_hbm.at[l // LW, w, l % LW], idxb[b], isem[b])

    def start_gather(b):
        pltpu.make_async_copy(idx4_hbm.at[0, 0, 0], idxb[b], isem[b]).wait()
        pltpu.async_copy(item_hbm.at[idxb[b]], gbuf[b], gsem[b])

    def drain_gather(b):
        pltpu.make_async_copy(item_hbm.at[pl.ds(0, BBLK)], gbuf[b],
                              gsem[b]).wait()

    def start_write(l, b):
        for dt in range(DT):
            pltpu.async_copy(obuf[b].at[dt], out_hbm.at[l, dt, w], wsem[b])

    def drain_write(b):
        for dt in range(DT):
            pltpu.make_async_copy(obuf[b].at[dt], out_hbm.at[0, dt, 0],
                                  wsem[b]).wait()

    def compute(l, b):
        row16 = lax.iota(jnp.int32, LANES)
        rows_j = [row16 + (j * LANES) for j in range(BBLK // LANES)]

        lvec = jnp.full((LANES,), l, jnp.int32)

        def d_body(d):
            dt = d // DW
            dw = d % DW
            cols = jnp.full((LANES,), d, jnp.int32)
            pvec = plsc.load_gather(pos_v, [lvec, cols])
            for j in range(BBLK // LANES):
                vals = plsc.load_gather(gbuf[b], [rows_j[j], cols])
                obuf[b][dt, dw, pl.ds(j * LANES, LANES)] = vals * SCALE + pvec

        plsc.parallel_loop(0, D, 1, unroll=2)(d_body)

    for p in range(NBUF):
        start_idx(p, p)
    for p in range(NBUF - 1):
        start_gather(p)

    def step(l, b, o):
        @pl.when(l + NBUF - 1 < L)
        def _():
            start_gather((b + NBUF - 1) % NBUF)

        drain_gather(b)

        @pl.when(o > 0)
        def _():
            drain_write(b)

        compute(l, b)
        start_write(l, b)

        @pl.when(l + NBUF < L)
        def _():
            start_idx(l + NBUF, b)

    def outer(o, carry):
        for b in range(NBUF):
            step(o * NBUF + b, b, o)
        return carry

    lax.fori_loop(0, L // NBUF, outer, 0)

    REM = L % NBUF
    for t in range(REM):
        l = L - REM + t
        b = l % NBUF
        drain_gather(b)
        drain_write(b)
        compute(l, b)
        start_write(l, b)

    for b in range(NBUF):
        drain_write(b)


@jax.jit
def _sc_embed(log_seqs, item_emb, pos_emb):
    idx4 = log_seqs.reshape(B // 128, 128, LT, LW).transpose(2, 0, 3, 1)
    kern = functools.partial(
        pl.kernel,
        out_type=jax.ShapeDtypeStruct((L, DT, NW, DW, 128), jnp.float32),
        mesh=plsc.VectorSubcoreMesh(core_axis_name="c", subcore_axis_name="s"),
        compiler_params=pltpu.CompilerParams(use_tc_tiling_on_sc=False,
                                             needs_layout_passes=False),
        scratch_types=[
            pltpu.VMEM((L, D), jnp.float32),
            pltpu.VMEM((128,), jnp.int32),
            pltpu.VMEM((128,), jnp.int32),
            pltpu.VMEM((128,), jnp.int32),
            pltpu.VMEM((BBLK, D), jnp.float32),
            pltpu.VMEM((BBLK, D), jnp.float32),
            pltpu.VMEM((BBLK, D), jnp.float32),
            pltpu.VMEM((DT, DW, 128), jnp.float32),
            pltpu.VMEM((DT, DW, 128), jnp.float32),
            pltpu.VMEM((DT, DW, 128), jnp.float32),
            pltpu.SemaphoreType.DMA,
            pltpu.SemaphoreType.DMA,
            pltpu.SemaphoreType.DMA,
            pltpu.SemaphoreType.DMA,
            pltpu.SemaphoreType.DMA,
            pltpu.SemaphoreType.DMA,
            pltpu.SemaphoreType.DMA,
            pltpu.SemaphoreType.DMA,
            pltpu.SemaphoreType.DMA,
        ],
    )(_sc_embed_body)
    out5 = kern(idx4, item_emb, pos_emb)
    return out5.transpose(2, 4, 0, 1, 3).reshape(B, L, D)


def _mask_body(seq_ref, mask_ref):
    mask_ref[...] = seq_ref[...] == PAD


@jax.jit
def _tc_mask(log_seqs):
    return pl.pallas_call(
        _mask_body,
        out_shape=jax.ShapeDtypeStruct((B, L), jnp.bool_),
    )(log_seqs)


def kernel(log_seqs, item_emb, pos_emb):
    log_seqs = log_seqs.astype(jnp.int32)
    seqs = _sc_embed(log_seqs, item_emb, pos_emb)
    mask = _tc_mask(log_seqs)
    return seqs, mask
